# baseline (speedup 1.0000x reference)
import functools

import jax
import jax.numpy as jnp
from jax import lax
from jax.experimental import pallas as pl
from jax.experimental.pallas import tpu as pltpu
from jax.experimental.pallas import tpu_sc as plsc

NEG = 0.2
NC, NS, L = 2, 16, 16
NW = NC * NS
CHUNK = 80
TW = 144
D = 128

_SC_PARAMS = pltpu.CompilerParams(
    use_tc_tiling_on_sc=False, needs_layout_passes=False)


def _leaky(v):
    return jnp.where(v >= 0, v, v * NEG)


def _tc1_body(x_ref, w_ref, cs_ref, cd_ref, t_ref, d_ref):
    h = jnp.dot(x_ref[...], w_ref[...], preferred_element_type=jnp.float32)
    t_ref[:, 0:D] = h
    t_ref[:, D:TW] = jnp.dot(h, cs_ref[...], preferred_element_type=jnp.float32)
    d_ref[...] = jnp.dot(h, cd_ref[...], preferred_element_type=jnp.float32)


def _tc2_body(a0, a1, t1, d1, b1r, wcatr, cs_ref, cd_ref, t_ref, d_ref):
    exs = jnp.exp(_leaky(t1[:, D:D + 1] + d1[:, 0:1]))
    num = a0[:, 0:D] + a1[:, 0:D] + t1[:, 0:D] * exs
    den = a0[:, D:D + 1] + a1[:, D:D + 1] + exs
    h = jnp.maximum(num / den + b1r[...], 0.0)
    h2 = jnp.dot(h, wcatr[...], preferred_element_type=jnp.float32)
    t_ref[:, 0:D] = h2
    t_ref[:, D:TW] = jnp.dot(h2, cs_ref[...], preferred_element_type=jnp.float32)
    d_ref[...] = jnp.dot(h2, cd_ref[...], preferred_element_type=jnp.float32)


def _tc3_body(a0, a1, t2, d2, bmr, blr, mu_ref, ls_ref):
    exm = jnp.exp(_leaky(t2[:, D:D + 1] + d2[:, 0:1]))
    exl = jnp.exp(_leaky(t2[:, D + 1:D + 2] + d2[:, 1:2]))
    mu_ref[...] = (a0[:, 0:64] + a1[:, 0:64] + t2[:, 0:64] * exm) / (
        a0[:, D:D + 1] + a1[:, D:D + 1] + exm) + bmr[...]
    ls_ref[...] = (a0[:, 64:D] + a1[:, 64:D] + t2[:, 64:D] * exl) / (
        a0[:, D + 1:D + 2] + a1[:, D + 1:D + 2] + exl) + blr[...]


def _sc_edge_pass(tab, dtab, src, dst, two_ex):
    n = tab.shape[0]
    e = src.shape[0]
    ept = e // NW
    nstripe = n // NS
    mesh = plsc.VectorSubcoreMesh(core_axis_name="c", subcore_axis_name="s",
                                  num_cores=NC, num_subcores=NS)

    @functools.partial(
        pl.kernel,
        out_type=jax.ShapeDtypeStruct((NC, n, TW), jnp.float32),
        mesh=mesh,
        compiler_params=_SC_PARAMS,
        scratch_types=[
            pltpu.VMEM((2, CHUNK), jnp.int32),
            pltpu.VMEM((2, CHUNK), jnp.int32),
            pltpu.VMEM((CHUNK,), jnp.float32),
            pltpu.VMEM((CHUNK,), jnp.float32),
            pltpu.VMEM((2, CHUNK, TW), jnp.float32),
            pltpu.VMEM((CHUNK, TW), jnp.float32),
            pltpu.VMEM((2, CHUNK, L), jnp.float32),
            pltpu.VMEM_SHARED((n, TW), jnp.float32),
            pltpu.SemaphoreType.DMA,
            pltpu.SemaphoreType.DMA,
        ],
    )
    def k(tab_hbm, dtab_hbm, src_hbm, dst_hbm, acc_hbm,
          src_b, dst_b, exa_buf, exb_buf, rows_in, rows_out, rows_dst,
          acc_sh, semR, semD):
        c = lax.axis_index("c")
        s = lax.axis_index("s")
        io16 = lax.iota(jnp.int32, L)
        oneh0 = jnp.where(io16 == 0, 1.0, 0.0).astype(jnp.float32)
        oneh1 = jnp.where(io16 == 1, 1.0, 0.0).astype(jnp.float32)

        def zrow(i, carry):
            for j in range(TW // L):
                rows_out[i, pl.ds(j * L, L)] = jnp.zeros((L,), jnp.float32)
            return carry
        lax.fori_loop(0, CHUNK, zrow, 0)
        base_row = s * nstripe
        nfull, rem = nstripe // CHUNK, nstripe % CHUNK
        for k2 in range(nfull):
            pltpu.sync_copy(rows_out, acc_sh.at[pl.ds(base_row + k2 * CHUNK, CHUNK)])
        if rem:
            pltpu.sync_copy(rows_out.at[pl.ds(0, rem)],
                            acc_sh.at[pl.ds(base_row + nfull * CHUNK, rem)])
        plsc.subcore_barrier()

        base_e = (c * NS + s) * ept
        nch = ept // CHUNK

        def sync_idx(kk, slot):
            eb = base_e + kk * CHUNK
            pltpu.sync_copy(src_hbm.at[pl.ds(eb, CHUNK)], src_b.at[slot])
            pltpu.sync_copy(dst_hbm.at[pl.ds(eb, CHUNK)], dst_b.at[slot])

        def issue_gather(slot, b):
            pltpu.async_copy(tab_hbm.at[src_b.at[slot]], rows_in.at[b], semR)
            pltpu.async_copy(dtab_hbm.at[dst_b.at[slot]], rows_dst.at[b], semD)

        def wait_gather():
            pltpu.make_async_copy(
                tab_hbm.at[src_b.at[0]], rows_in.at[0], semR).wait()
            pltpu.make_async_copy(
                dtab_hbm.at[dst_b.at[0]], rows_dst.at[0], semD).wait()

        def compute_scatter(b):
            bb = jnp.full((L,), b, jnp.int32)
            for g in range(CHUNK // L):
                ridx = io16 + g * L
                asv = plsc.load_gather(
                    rows_in, [bb, ridx, jnp.full((L,), D, jnp.int32)])
                adv = plsc.load_gather(
                    rows_dst, [bb, ridx, jnp.zeros((L,), jnp.int32)])
                exa_buf[pl.ds(g * L, L)] = jnp.exp(_leaky(asv + adv))
                if two_ex:
                    aslv = plsc.load_gather(
                        rows_in, [bb, ridx, jnp.full((L,), D + 1, jnp.int32)])
                    adlv = plsc.load_gather(
                        rows_dst, [bb, ridx, jnp.full((L,), 1, jnp.int32)])
                    exb_buf[pl.ds(g * L, L)] = jnp.exp(_leaky(aslv + adlv))

            def r_body(g, cr):
                exa16 = exa_buf[pl.ds(g * L, L)]
                exb16 = exb_buf[pl.ds(g * L, L)] if two_ex else exa16
                for m in range(L):
                    i = g * L + m
                    exa = exa16[m]
                    if two_ex:
                        exb = exb16[m]
                        for j in range(D // (2 * L)):
                            rows_out[i, pl.ds(j * L, L)] = (
                                rows_in[b, i, pl.ds(j * L, L)] * exa)
                        for j in range(D // (2 * L), D // L):
                            rows_out[i, pl.ds(j * L, L)] = (
                                rows_in[b, i, pl.ds(j * L, L)] * exb)
                        rows_out[i, pl.ds(D, L)] = oneh0 * exa + oneh1 * exb
                    else:
                        for j in range(D // L):
                            rows_out[i, pl.ds(j * L, L)] = (
                                rows_in[b, i, pl.ds(j * L, L)] * exa)
                        rows_out[i, pl.ds(D, L)] = oneh0 * exa
                return cr
            lax.fori_loop(0, CHUNK // L, r_body, 0)
            pltpu.sync_copy(rows_out, acc_sh.at[dst_b.at[b]], add=True)

        sync_idx(0, 0)
        issue_gather(0, 0)

        def chunk_body(kk, carry):
            b = lax.rem(kk, 2)
            nb = 1 - b
            wait_gather()
            sync_idx(kk + 1, nb)
            issue_gather(nb, nb)
            compute_scatter(b)
            return carry

        lax.fori_loop(0, nch - 1, chunk_body, 0)
        wait_gather()
        compute_scatter((nch - 1) % 2)
        plsc.subcore_barrier()

        for k2 in range(nfull):
            r0 = base_row + k2 * CHUNK
            pltpu.sync_copy(acc_sh.at[pl.ds(r0, CHUNK)], rows_out)
            pltpu.sync_copy(rows_out, acc_hbm.at[c, pl.ds(r0, CHUNK)])
        if rem:
            r0 = base_row + nfull * CHUNK
            pltpu.sync_copy(acc_sh.at[pl.ds(r0, rem)], rows_out.at[pl.ds(0, rem)])
            pltpu.sync_copy(rows_out.at[pl.ds(0, rem)], acc_hbm.at[c, pl.ds(r0, rem)])

    return k(tab, dtab, src, dst)


def kernel(x, edge_index, W1, a_s1, a_d1, b1,
           W_mu, a_s_mu, a_d_mu, b_mu, W_ls, a_s_ls, a_d_ls, b_ls):
    n, d_in = x.shape
    d_hid = W1.shape[1]
    d_out = W_mu.shape[1]
    src = edge_index[0]
    dst = edge_index[1]
    f32 = jnp.float32

    C1s = jnp.zeros((d_hid, L), f32).at[:, 0].set(a_s1[0])
    C1d = jnp.zeros((d_hid, L), f32).at[:, 0].set(a_d1[0])
    t1, dt1 = pl.pallas_call(
        _tc1_body,
        out_shape=[jax.ShapeDtypeStruct((n, TW), f32),
                   jax.ShapeDtypeStruct((n, L), f32)],
    )(x, W1, C1s, C1d)

    acc1 = _sc_edge_pass(t1, dt1, src, dst, two_ex=False)

    Wcat = jnp.concatenate([W_mu, W_ls], axis=1)
    C2s = (jnp.zeros((2 * d_out, L), f32)
           .at[:d_out, 0].set(a_s_mu[0]).at[d_out:, 1].set(a_s_ls[0]))
    C2d = (jnp.zeros((2 * d_out, L), f32)
           .at[:d_out, 0].set(a_d_mu[0]).at[d_out:, 1].set(a_d_ls[0]))
    t2, dt2 = pl.pallas_call(
        _tc2_body,
        out_shape=[jax.ShapeDtypeStruct((n, TW), f32),
                   jax.ShapeDtypeStruct((n, L), f32)],
    )(acc1[0], acc1[1], t1, dt1, b1.reshape(1, d_hid), Wcat, C2s, C2d)

    acc2 = _sc_edge_pass(t2, dt2, src, dst, two_ex=True)

    mu, logstd = pl.pallas_call(
        _tc3_body,
        out_shape=[jax.ShapeDtypeStruct((n, d_out), f32)] * 2,
    )(acc2[0], acc2[1], t2, dt2, b_mu.reshape(1, d_out),
      b_ls.reshape(1, d_out))
    return (mu, logstd)


# speedup vs baseline: 1.7171x; 1.7171x over previous
import functools

import jax
import jax.numpy as jnp
from jax import lax
from jax.experimental import pallas as pl
from jax.experimental.pallas import tpu as pltpu
from jax.experimental.pallas import tpu_sc as plsc

NEG = 0.2
NC, NS, L = 2, 16, 16
NW = NC * NS
CHUNK = 80
TW = 144
D = 128

_SC_PARAMS = pltpu.CompilerParams(
    use_tc_tiling_on_sc=False, needs_layout_passes=False)


def _leaky(v):
    return jnp.where(v >= 0, v, v * NEG)


def _tc1_body(x_ref, w_ref, cs_ref, cd_ref, t_ref, d_ref):
    h = jnp.dot(x_ref[...], w_ref[...], preferred_element_type=jnp.float32)
    t_ref[:, 0:D] = h
    t_ref[:, D:TW] = jnp.dot(h, cs_ref[...], preferred_element_type=jnp.float32)
    d_ref[...] = jnp.dot(h, cd_ref[...], preferred_element_type=jnp.float32)


def _tc2_body(a0, a1, t1, d1, b1r, wcatr, cs_ref, cd_ref, t_ref, d_ref):
    exs = jnp.exp(_leaky(t1[:, D:D + 1] + d1[:, 0:1]))
    num = a0[:, 0:D] + a1[:, 0:D] + t1[:, 0:D] * exs
    den = a0[:, D:D + 1] + a1[:, D:D + 1] + exs
    h = jnp.maximum(num / den + b1r[...], 0.0)
    h2 = jnp.dot(h, wcatr[...], preferred_element_type=jnp.float32)
    t_ref[:, 0:D] = h2
    t_ref[:, D:TW] = jnp.dot(h2, cs_ref[...], preferred_element_type=jnp.float32)
    d_ref[...] = jnp.dot(h2, cd_ref[...], preferred_element_type=jnp.float32)


def _tc3_body(a0, a1, t2, d2, bmr, blr, mu_ref, ls_ref):
    exm = jnp.exp(_leaky(t2[:, D:D + 1] + d2[:, 0:1]))
    exl = jnp.exp(_leaky(t2[:, D + 1:D + 2] + d2[:, 1:2]))
    mu_ref[...] = (a0[:, 0:64] + a1[:, 0:64] + t2[:, 0:64] * exm) / (
        a0[:, D:D + 1] + a1[:, D:D + 1] + exm) + bmr[...]
    ls_ref[...] = (a0[:, 64:D] + a1[:, 64:D] + t2[:, 64:D] * exl) / (
        a0[:, D + 1:D + 2] + a1[:, D + 1:D + 2] + exl) + blr[...]


def _sc_edge_pass(tab, dtab, src, dst, two_ex):
    n = tab.shape[0]
    e = src.shape[0]
    ept = e // NW
    nstripe = n // NS
    mesh = plsc.VectorSubcoreMesh(core_axis_name="c", subcore_axis_name="s",
                                  num_cores=NC, num_subcores=NS)

    @functools.partial(
        pl.kernel,
        out_type=jax.ShapeDtypeStruct((NC, n, TW), jnp.float32),
        mesh=mesh,
        compiler_params=_SC_PARAMS,
        scratch_types=[
            pltpu.VMEM((2, CHUNK), jnp.int32),
            pltpu.VMEM((2, CHUNK), jnp.int32),
            pltpu.VMEM((CHUNK,), jnp.float32),
            pltpu.VMEM((CHUNK,), jnp.float32),
            pltpu.VMEM((2, CHUNK, TW), jnp.float32),
            pltpu.VMEM((CHUNK, TW), jnp.float32),
            pltpu.VMEM((2, CHUNK, L), jnp.float32),
            pltpu.VMEM_SHARED((n, TW), jnp.float32),
            pltpu.SemaphoreType.DMA,
            pltpu.SemaphoreType.DMA,
        ],
    )
    def k(tab_hbm, dtab_hbm, src_hbm, dst_hbm, acc_hbm,
          src_b, dst_b, exa_buf, exb_buf, rows_in, rows_out, rows_dst,
          acc_sh, semR, semD):
        c = lax.axis_index("c")
        s = lax.axis_index("s")
        io16 = lax.iota(jnp.int32, L)
        oneh0 = jnp.where(io16 == 0, 1.0, 0.0).astype(jnp.float32)
        oneh1 = jnp.where(io16 == 1, 1.0, 0.0).astype(jnp.float32)

        def zrow(i, carry):
            for j in range(TW // L):
                rows_out[i, pl.ds(j * L, L)] = jnp.zeros((L,), jnp.float32)
            return carry
        lax.fori_loop(0, CHUNK, zrow, 0)
        base_row = s * nstripe
        nfull, rem = nstripe // CHUNK, nstripe % CHUNK
        for k2 in range(nfull):
            pltpu.sync_copy(rows_out, acc_sh.at[pl.ds(base_row + k2 * CHUNK, CHUNK)])
        if rem:
            pltpu.sync_copy(rows_out.at[pl.ds(0, rem)],
                            acc_sh.at[pl.ds(base_row + nfull * CHUNK, rem)])
        plsc.subcore_barrier()

        base_e = (c * NS + s) * ept
        nch = ept // CHUNK

        def sync_idx(kk, slot):
            eb = base_e + kk * CHUNK
            pltpu.sync_copy(src_hbm.at[pl.ds(eb, CHUNK)], src_b.at[slot])
            pltpu.sync_copy(dst_hbm.at[pl.ds(eb, CHUNK)], dst_b.at[slot])

        def issue_gather(b):
            cp = pltpu.async_copy(tab_hbm.at[src_b.at[b]], rows_in.at[b], semR)
            cp2 = pltpu.async_copy(dtab_hbm.at[dst_b.at[b]], rows_dst.at[b], semD)
            return cp, cp2

        def compute_scatter(b):
            bb = jnp.full((L,), b, jnp.int32)
            for g in range(CHUNK // L):
                ridx = io16 + g * L
                asv = plsc.load_gather(
                    rows_in, [bb, ridx, jnp.full((L,), D, jnp.int32)])
                adv = plsc.load_gather(
                    rows_dst, [bb, ridx, jnp.zeros((L,), jnp.int32)])
                exa_buf[pl.ds(g * L, L)] = jnp.exp(_leaky(asv + adv))
                if two_ex:
                    aslv = plsc.load_gather(
                        rows_in, [bb, ridx, jnp.full((L,), D + 1, jnp.int32)])
                    adlv = plsc.load_gather(
                        rows_dst, [bb, ridx, jnp.full((L,), 1, jnp.int32)])
                    exb_buf[pl.ds(g * L, L)] = jnp.exp(_leaky(aslv + adlv))

            def r_body(g, cr):
                exa16 = exa_buf[pl.ds(g * L, L)]
                exb16 = exb_buf[pl.ds(g * L, L)] if two_ex else exa16
                for m in range(L):
                    i = g * L + m
                    exa = exa16[m]
                    if two_ex:
                        exb = exb16[m]
                        for j in range(D // (2 * L)):
                            rows_out[i, pl.ds(j * L, L)] = (
                                rows_in[b, i, pl.ds(j * L, L)] * exa)
                        for j in range(D // (2 * L), D // L):
                            rows_out[i, pl.ds(j * L, L)] = (
                                rows_in[b, i, pl.ds(j * L, L)] * exb)
                        rows_out[i, pl.ds(D, L)] = oneh0 * exa + oneh1 * exb
                    else:
                        for j in range(D // L):
                            rows_out[i, pl.ds(j * L, L)] = (
                                rows_in[b, i, pl.ds(j * L, L)] * exa)
                        rows_out[i, pl.ds(D, L)] = oneh0 * exa
                return cr
            lax.fori_loop(0, CHUNK // L, r_body, 0)
            pltpu.sync_copy(rows_out, acc_sh.at[dst_b.at[b]], add=True)

        def pair_body(kk2, carry):
            a = 2 * kk2
            sync_idx(a, 0)
            cpa, cpa2 = issue_gather(0)
            sync_idx(a + 1, 1)
            cpa.wait()
            cpa2.wait()
            cpb, cpb2 = issue_gather(1)
            compute_scatter(0)
            cpb.wait()
            cpb2.wait()
            compute_scatter(1)
            return carry

        lax.fori_loop(0, nch // 2, pair_body, 0)
        if nch % 2:
            sync_idx(nch - 1, 0)
            cpa, cpa2 = issue_gather(0)
            cpa.wait()
            cpa2.wait()
            compute_scatter(0)
        plsc.subcore_barrier()

        for k2 in range(nfull):
            r0 = base_row + k2 * CHUNK
            pltpu.sync_copy(acc_sh.at[pl.ds(r0, CHUNK)], rows_out)
            pltpu.sync_copy(rows_out, acc_hbm.at[c, pl.ds(r0, CHUNK)])
        if rem:
            r0 = base_row + nfull * CHUNK
            pltpu.sync_copy(acc_sh.at[pl.ds(r0, rem)], rows_out.at[pl.ds(0, rem)])
            pltpu.sync_copy(rows_out.at[pl.ds(0, rem)], acc_hbm.at[c, pl.ds(r0, rem)])

    return k(tab, dtab, src, dst)


def kernel(x, edge_index, W1, a_s1, a_d1, b1,
           W_mu, a_s_mu, a_d_mu, b_mu, W_ls, a_s_ls, a_d_ls, b_ls):
    n, d_in = x.shape
    d_hid = W1.shape[1]
    d_out = W_mu.shape[1]
    src = edge_index[0]
    dst = edge_index[1]
    f32 = jnp.float32

    C1s = jnp.zeros((d_hid, L), f32).at[:, 0].set(a_s1[0])
    C1d = jnp.zeros((d_hid, L), f32).at[:, 0].set(a_d1[0])
    t1, dt1 = pl.pallas_call(
        _tc1_body,
        out_shape=[jax.ShapeDtypeStruct((n, TW), f32),
                   jax.ShapeDtypeStruct((n, L), f32)],
    )(x, W1, C1s, C1d)

    acc1 = _sc_edge_pass(t1, dt1, src, dst, two_ex=False)

    Wcat = jnp.concatenate([W_mu, W_ls], axis=1)
    C2s = (jnp.zeros((2 * d_out, L), f32)
           .at[:d_out, 0].set(a_s_mu[0]).at[d_out:, 1].set(a_s_ls[0]))
    C2d = (jnp.zeros((2 * d_out, L), f32)
           .at[:d_out, 0].set(a_d_mu[0]).at[d_out:, 1].set(a_d_ls[0]))
    t2, dt2 = pl.pallas_call(
        _tc2_body,
        out_shape=[jax.ShapeDtypeStruct((n, TW), f32),
                   jax.ShapeDtypeStruct((n, L), f32)],
    )(acc1[0], acc1[1], t1, dt1, b1.reshape(1, d_hid), Wcat, C2s, C2d)

    acc2 = _sc_edge_pass(t2, dt2, src, dst, two_ex=True)

    mu, logstd = pl.pallas_call(
        _tc3_body,
        out_shape=[jax.ShapeDtypeStruct((n, d_out), f32)] * 2,
    )(acc2[0], acc2[1], t2, dt2, b_mu.reshape(1, d_out),
      b_ls.reshape(1, d_out))
    return (mu, logstd)


# speedup vs baseline: 1.8944x; 1.1033x over previous
import functools

import jax
import jax.numpy as jnp
from jax import lax
from jax.experimental import pallas as pl
from jax.experimental.pallas import tpu as pltpu
from jax.experimental.pallas import tpu_sc as plsc

NEG = 0.2
NC, NS, L = 2, 16, 16
NW = NC * NS
CHUNK = 40
UNROLL = 5
TW = 144
D = 128

_SC_PARAMS = pltpu.CompilerParams(
    use_tc_tiling_on_sc=False, needs_layout_passes=False)


def _leaky(v):
    return jnp.where(v >= 0, v, v * NEG)


def _tc1_body(x_ref, w_ref, cs_ref, cd_ref, t_ref, d_ref):
    h = jnp.dot(x_ref[...], w_ref[...], preferred_element_type=jnp.float32)
    t_ref[:, 0:D] = h
    t_ref[:, D:TW] = jnp.dot(h, cs_ref[...], preferred_element_type=jnp.float32)
    d_ref[...] = jnp.dot(h, cd_ref[...], preferred_element_type=jnp.float32)


def _tc2_body(a0, a1, t1, d1, b1r, wcatr, cs_ref, cd_ref, t_ref, d_ref):
    exs = jnp.exp(_leaky(t1[:, D:D + 1] + d1[:, 0:1]))
    num = a0[:, 0:D] + a1[:, 0:D] + t1[:, 0:D] * exs
    den = a0[:, D:D + 1] + a1[:, D:D + 1] + exs
    h = jnp.maximum(num / den + b1r[...], 0.0)
    h2 = jnp.dot(h, wcatr[...], preferred_element_type=jnp.float32)
    t_ref[:, 0:D] = h2
    t_ref[:, D:TW] = jnp.dot(h2, cs_ref[...], preferred_element_type=jnp.float32)
    d_ref[...] = jnp.dot(h2, cd_ref[...], preferred_element_type=jnp.float32)


def _tc3_body(a0, a1, t2, d2, bmr, blr, mu_ref, ls_ref):
    exm = jnp.exp(_leaky(t2[:, D:D + 1] + d2[:, 0:1]))
    exl = jnp.exp(_leaky(t2[:, D + 1:D + 2] + d2[:, 1:2]))
    mu_ref[...] = (a0[:, 0:64] + a1[:, 0:64] + t2[:, 0:64] * exm) / (
        a0[:, D:D + 1] + a1[:, D:D + 1] + exm) + bmr[...]
    ls_ref[...] = (a0[:, 64:D] + a1[:, 64:D] + t2[:, 64:D] * exl) / (
        a0[:, D + 1:D + 2] + a1[:, D + 1:D + 2] + exl) + blr[...]


def _sc_edge_pass(tab, dtab, src, dst, two_ex):
    n = tab.shape[0]
    e = src.shape[0]
    ept = e // NW
    nstripe = n // NS
    mesh = plsc.VectorSubcoreMesh(core_axis_name="c", subcore_axis_name="s",
                                  num_cores=NC, num_subcores=NS)

    @functools.partial(
        pl.kernel,
        out_type=jax.ShapeDtypeStruct((NC, n, TW), jnp.float32),
        mesh=mesh,
        compiler_params=_SC_PARAMS,
        scratch_types=[
            pltpu.VMEM((UNROLL, CHUNK), jnp.int32),
            pltpu.VMEM((UNROLL, CHUNK), jnp.int32),
            pltpu.VMEM((CHUNK,), jnp.float32),
            pltpu.VMEM((CHUNK,), jnp.float32),
            pltpu.VMEM((UNROLL, CHUNK, TW), jnp.float32),
            pltpu.VMEM((CHUNK, TW), jnp.float32),
            pltpu.VMEM((UNROLL, CHUNK, L), jnp.float32),
            pltpu.VMEM_SHARED((n, TW), jnp.float32),
        ] + [pltpu.SemaphoreType.DMA] * UNROLL,
    )
    def k(tab_hbm, dtab_hbm, src_hbm, dst_hbm, acc_hbm,
          src_b, dst_b, exa_buf, exb_buf, rows_in, rows_out, rows_dst,
          acc_sh, *sems):
        c = lax.axis_index("c")
        s = lax.axis_index("s")
        io16 = lax.iota(jnp.int32, L)
        oneh0 = jnp.where(io16 == 0, 1.0, 0.0).astype(jnp.float32)
        oneh1 = jnp.where(io16 == 1, 1.0, 0.0).astype(jnp.float32)

        def zrow(i, carry):
            for j in range(TW // L):
                rows_out[i, pl.ds(j * L, L)] = jnp.zeros((L,), jnp.float32)
            return carry
        lax.fori_loop(0, CHUNK, zrow, 0)
        base_row = s * nstripe
        nfull, rem = nstripe // CHUNK, nstripe % CHUNK
        for k2 in range(nfull):
            pltpu.sync_copy(rows_out, acc_sh.at[pl.ds(base_row + k2 * CHUNK, CHUNK)])
        if rem:
            pltpu.sync_copy(rows_out.at[pl.ds(0, rem)],
                            acc_sh.at[pl.ds(base_row + nfull * CHUNK, rem)])
        plsc.subcore_barrier()

        base_e = (c * NS + s) * ept
        nch = ept // CHUNK

        def issue_idx(kk, slot):
            eb = base_e + kk * CHUNK
            c1 = pltpu.async_copy(
                src_hbm.at[pl.ds(eb, CHUNK)], src_b.at[slot], sems[slot])
            c2 = pltpu.async_copy(
                dst_hbm.at[pl.ds(eb, CHUNK)], dst_b.at[slot], sems[slot])
            return c1, c2

        def issue_gather(b):
            cp = pltpu.async_copy(
                tab_hbm.at[src_b.at[b]], rows_in.at[b], sems[b])
            cp2 = pltpu.async_copy(
                dtab_hbm.at[dst_b.at[b]], rows_dst.at[b], sems[b])
            return cp, cp2

        def compute_scatter(b):
            bb = jnp.full((L,), b, jnp.int32)
            for g in range(CHUNK // L):
                ridx = io16 + g * L
                asv = plsc.load_gather(
                    rows_in, [bb, ridx, jnp.full((L,), D, jnp.int32)])
                adv = plsc.load_gather(
                    rows_dst, [bb, ridx, jnp.zeros((L,), jnp.int32)])
                exa_buf[pl.ds(g * L, L)] = jnp.exp(_leaky(asv + adv))
                if two_ex:
                    aslv = plsc.load_gather(
                        rows_in, [bb, ridx, jnp.full((L,), D + 1, jnp.int32)])
                    adlv = plsc.load_gather(
                        rows_dst, [bb, ridx, jnp.full((L,), 1, jnp.int32)])
                    exb_buf[pl.ds(g * L, L)] = jnp.exp(_leaky(aslv + adlv))

            def r_body(g, cr):
                exa16 = exa_buf[pl.ds(g * L, L)]
                exb16 = exb_buf[pl.ds(g * L, L)] if two_ex else exa16
                for m in range(L):
                    i = g * L + m
                    exa = exa16[m]
                    if two_ex:
                        exb = exb16[m]
                        for j in range(D // (2 * L)):
                            rows_out[i, pl.ds(j * L, L)] = (
                                rows_in[b, i, pl.ds(j * L, L)] * exa)
                        for j in range(D // (2 * L), D // L):
                            rows_out[i, pl.ds(j * L, L)] = (
                                rows_in[b, i, pl.ds(j * L, L)] * exb)
                        rows_out[i, pl.ds(D, L)] = oneh0 * exa + oneh1 * exb
                    else:
                        for j in range(D // L):
                            rows_out[i, pl.ds(j * L, L)] = (
                                rows_in[b, i, pl.ds(j * L, L)] * exa)
                        rows_out[i, pl.ds(D, L)] = oneh0 * exa
                return cr
            lax.fori_loop(0, CHUNK // L, r_body, 0)
            pltpu.sync_copy(rows_out, acc_sh.at[dst_b.at[b]], add=True)

        assert nch % UNROLL == 0
        def group_body(q, carry):
            a = q * UNROLL
            idx_descs = [issue_idx(a + j, j) for j in range(UNROLL)]
            g_descs = []
            for j in range(UNROLL):
                idx_descs[j][0].wait()
                idx_descs[j][1].wait()
                g_descs.append(issue_gather(j))
            for j in range(UNROLL):
                g_descs[j][0].wait()
                g_descs[j][1].wait()
                compute_scatter(j)
            return carry

        lax.fori_loop(0, nch // UNROLL, group_body, 0)
        plsc.subcore_barrier()

        for k2 in range(nfull):
            r0 = base_row + k2 * CHUNK
            pltpu.sync_copy(acc_sh.at[pl.ds(r0, CHUNK)], rows_out)
            pltpu.sync_copy(rows_out, acc_hbm.at[c, pl.ds(r0, CHUNK)])
        if rem:
            r0 = base_row + nfull * CHUNK
            pltpu.sync_copy(acc_sh.at[pl.ds(r0, rem)], rows_out.at[pl.ds(0, rem)])
            pltpu.sync_copy(rows_out.at[pl.ds(0, rem)], acc_hbm.at[c, pl.ds(r0, rem)])

    return k(tab, dtab, src, dst)


def kernel(x, edge_index, W1, a_s1, a_d1, b1,
           W_mu, a_s_mu, a_d_mu, b_mu, W_ls, a_s_ls, a_d_ls, b_ls):
    n, d_in = x.shape
    d_hid = W1.shape[1]
    d_out = W_mu.shape[1]
    src = edge_index[0]
    dst = edge_index[1]
    f32 = jnp.float32

    C1s = jnp.zeros((d_hid, L), f32).at[:, 0].set(a_s1[0])
    C1d = jnp.zeros((d_hid, L), f32).at[:, 0].set(a_d1[0])
    t1, dt1 = pl.pallas_call(
        _tc1_body,
        out_shape=[jax.ShapeDtypeStruct((n, TW), f32),
                   jax.ShapeDtypeStruct((n, L), f32)],
    )(x, W1, C1s, C1d)

    acc1 = _sc_edge_pass(t1, dt1, src, dst, two_ex=False)

    Wcat = jnp.concatenate([W_mu, W_ls], axis=1)
    C2s = (jnp.zeros((2 * d_out, L), f32)
           .at[:d_out, 0].set(a_s_mu[0]).at[d_out:, 1].set(a_s_ls[0]))
    C2d = (jnp.zeros((2 * d_out, L), f32)
           .at[:d_out, 0].set(a_d_mu[0]).at[d_out:, 1].set(a_d_ls[0]))
    t2, dt2 = pl.pallas_call(
        _tc2_body,
        out_shape=[jax.ShapeDtypeStruct((n, TW), f32),
                   jax.ShapeDtypeStruct((n, L), f32)],
    )(acc1[0], acc1[1], t1, dt1, b1.reshape(1, d_hid), Wcat, C2s, C2d)

    acc2 = _sc_edge_pass(t2, dt2, src, dst, two_ex=True)

    mu, logstd = pl.pallas_call(
        _tc3_body,
        out_shape=[jax.ShapeDtypeStruct((n, d_out), f32)] * 2,
    )(acc2[0], acc2[1], t2, dt2, b_mu.reshape(1, d_out),
      b_ls.reshape(1, d_out))
    return (mu, logstd)
